# baseline (speedup 1.0000x reference)
import functools

import jax
import jax.numpy as jnp
from jax import lax
from jax.experimental import pallas as pl
from jax.experimental.pallas import tpu as pltpu
from jax.experimental.pallas import tpu_sc as plsc

_NUM_CORES = 2
_NUM_SUBCORES = 16
_NUM_WORKERS = _NUM_CORES * _NUM_SUBCORES
_CHUNK = 64


def kernel(inputs, input_table, position_table):
    batch, seqlen = inputs.shape
    vocab, embed = input_table.shape
    n = batch * seqlen
    rpw = n // _NUM_WORKERS
    n_chunks = rpw // _CHUNK
    wpb = seqlen // rpw

    mesh = plsc.VectorSubcoreMesh(
        core_axis_name="c", subcore_axis_name="s",
        num_cores=_NUM_CORES, num_subcores=_NUM_SUBCORES)

    @functools.partial(
        pl.kernel,
        out_type=jax.ShapeDtypeStruct((batch, seqlen, embed), jnp.float32),
        mesh=mesh,
        scratch_types=[
            pltpu.VMEM((rpw,), jnp.int32),
            pltpu.VMEM((n_chunks, _CHUNK), jnp.int32),
            pltpu.VMEM((rpw, embed), jnp.float32),
            pltpu.VMEM_SHARED((_NUM_SUBCORES * rpw, embed), jnp.float32),
            pltpu.SemaphoreType.DMA,
            pltpu.SemaphoreType.DMA,
            pltpu.SemaphoreType.DMA,
        ],
    )
    def emb_kernel(idx_hbm, tab_hbm, pos_hbm, out_hbm,
                   idx_v, scat_v, rows_v, shared, sem, sem2, sem3):
        c = lax.axis_index("c")
        s = lax.axis_index("s")
        wid = s * _NUM_CORES + c
        b = wid // wpb
        col = (wid % wpb) * rpw
        my_shared = shared.at[pl.ds(s * rpw, rpw)]
        cp_pos = pltpu.async_copy(pos_hbm.at[pl.ds(col, rpw)], my_shared,
                                  sem2)
        cp_idx = pltpu.async_copy(idx_hbm.at[b, pl.ds(col, rpw)], idx_v, sem)
        lanes = lax.iota(jnp.int32, 16)
        for j in range(n_chunks):
            for k in range(_CHUNK // 16):
                scat_v[j, pl.ds(k * 16, 16)] = lanes + (
                    s * rpw + j * _CHUNK + k * 16)
        with jax.named_scope("wait_idx"):
            cp_idx.wait()
        with jax.named_scope("issue_gathers"):
            gathers = [
                pltpu.async_copy(
                    tab_hbm.at[idx_v.at[pl.ds(j * _CHUNK, _CHUNK)]],
                    rows_v.at[pl.ds(j * _CHUNK, _CHUNK)], sem)
                for j in range(n_chunks)
            ]
        with jax.named_scope("wait_pos"):
            cp_pos.wait()
        outs = []
        for j in range(n_chunks):
            with jax.named_scope(f"wait_gather{j}"):
                gathers[j].wait()
            with jax.named_scope(f"sadd{j}"):
                pltpu.sync_copy(
                    rows_v.at[pl.ds(j * _CHUNK, _CHUNK)],
                    shared.at[scat_v.at[j]], add=True)
            outs.append(pltpu.async_copy(
                shared.at[pl.ds(s * rpw + j * _CHUNK, _CHUNK)],
                out_hbm.at[b, pl.ds(col + j * _CHUNK, _CHUNK)], sem3))
        with jax.named_scope("wait_outs"):
            for o in outs:
                o.wait()

    return emb_kernel(inputs, input_table, position_table)


# speedup vs baseline: 1.0062x; 1.0062x over previous
import functools

import jax
import jax.numpy as jnp
from jax import lax
from jax.experimental import pallas as pl
from jax.experimental.pallas import tpu as pltpu
from jax.experimental.pallas import tpu_sc as plsc

_NUM_CORES = 2
_NUM_SUBCORES = 16
_NUM_WORKERS = _NUM_CORES * _NUM_SUBCORES


def kernel(inputs, input_table, position_table):
    batch, seqlen = inputs.shape
    vocab, embed = input_table.shape
    chunk = seqlen // _NUM_WORKERS
    rpw = batch * chunk

    mesh = plsc.VectorSubcoreMesh(
        core_axis_name="c", subcore_axis_name="s",
        num_cores=_NUM_CORES, num_subcores=_NUM_SUBCORES)

    @functools.partial(
        pl.kernel,
        out_type=jax.ShapeDtypeStruct((batch, seqlen, embed), jnp.float32),
        mesh=mesh,
        scratch_types=[
            pltpu.VMEM((batch, chunk), jnp.int32),
            pltpu.VMEM((batch, chunk), jnp.int32),
            pltpu.VMEM((chunk, embed), jnp.float32),
            pltpu.VMEM((rpw, embed), jnp.float32),
            pltpu.VMEM_SHARED((_NUM_SUBCORES * rpw, embed), jnp.float32),
            pltpu.SemaphoreType.DMA,
            pltpu.SemaphoreType.DMA,
            pltpu.SemaphoreType.DMA,
            pltpu.SemaphoreType.DMA,
            pltpu.SemaphoreType.DMA,
        ],
    )
    def emb_kernel(idx_hbm, tab_hbm, pos_hbm, out_hbm,
                   idx_v, scat_v, pos_v, rows_v, shared,
                   sem_p, sem_i, sem_r, sem_a, sem_o):
        c = lax.axis_index("c")
        s = lax.axis_index("s")
        wid = s * _NUM_CORES + c
        col = wid * chunk
        base = s * rpw
        cp_pos = pltpu.async_copy(pos_hbm.at[pl.ds(col, chunk)], pos_v,
                                  sem_p)
        cp_idx = [
            pltpu.async_copy(idx_hbm.at[b, pl.ds(col, chunk)], idx_v.at[b],
                             sem_i)
            for b in range(batch)
        ]
        lanes = lax.iota(jnp.int32, 16)
        for b in range(batch):
            for k in range(chunk // 16):
                scat_v[b, pl.ds(k * 16, 16)] = lanes + (
                    base + b * chunk + k * 16)
        cp_pos.wait()
        reps = [
            pltpu.async_copy(
                pos_v, shared.at[pl.ds(base + b * chunk, chunk)], sem_r)
            for b in range(batch)
        ]
        gathers = []
        for b in range(batch):
            cp_idx[b].wait()
            gathers.append(pltpu.async_copy(
                tab_hbm.at[idx_v.at[b]],
                rows_v.at[pl.ds(b * chunk, chunk)], sem_i))
        for r in reps:
            r.wait()
        sadds = []
        for b in range(batch):
            gathers[b].wait()
            sadds.append(pltpu.async_copy(
                rows_v.at[pl.ds(b * chunk, chunk)],
                shared.at[scat_v.at[b]], sem_a, add=True))
        outs = []
        for b in range(batch):
            sadds[b].wait()
            outs.append(pltpu.async_copy(
                shared.at[pl.ds(base + b * chunk, chunk)],
                out_hbm.at[b, pl.ds(col, chunk)], sem_o))
        for o in outs:
            o.wait()

    return emb_kernel(inputs, input_table, position_table)
